# baseline (speedup 1.0000x reference)
import functools
import math

import jax
import jax.numpy as jnp
import numpy as np
from jax import lax
from jax.experimental import pallas as pl

_T = 2048
_PRED = 256
_K = 16
_M = 1024


_TH = 1032


def _basis():
    tau = np.arange(_TH, dtype=np.int64)[:, None]
    m = np.arange(1, _M + 1, dtype=np.int64)[None, :]
    ang = 2.0 * np.pi * ((tau * m) % _T).astype(np.float64) / _T
    c = np.cos(ang)
    s = np.sin(ang)
    c[:, -1] = 0.0
    s[:, -1] = 0.0
    c[_T // 2 + 1:, :] = 0.0
    s[_T // 2 + 1:, :] = 0.0
    return c.astype(np.float32), s.astype(np.float32)


_C_TABLE, _S_TABLE = _basis()


def _body(x_ref, xf_ref, c_ref, s_ref, o_ref):
    xb = jnp.concatenate([x_ref[0], x_ref[1]], axis=1)
    xfb = jnp.concatenate([xf_ref[0], xf_ref[1]], axis=1)
    cb = c_ref[...]
    sb = s_ref[...]
    n = xb.shape[1]
    xr = xfb[0:1023, :]
    xmid = xb[1:1024, :]
    zpad = jnp.zeros((7, n), jnp.float32)
    zrow = jnp.zeros((1, n), jnp.float32)
    xe = jnp.concatenate([xb[0:1, :], xmid + xr, xb[1024:1025, :], zpad], 0)
    xo = jnp.concatenate([zrow, xmid - xr, zrow, zpad], 0)
    hi = jax.lax.Precision.HIGHEST
    dn_fwd = (((0,), (0,)), ((), ()))
    re = lax.dot_general(cb, xe, dn_fwd, precision=hi,
                         preferred_element_type=jnp.float32)
    im = -lax.dot_general(sb, xo, dn_fwd, precision=hi,
                          preferred_element_type=jnp.float32)
    mag2 = re * re + im * im

    n = mag2.shape[1]
    iota = lax.broadcasted_iota(jnp.int32, (_M, n), 0)
    work = mag2
    sel = jnp.zeros(mag2.shape, jnp.float32)
    for _ in range(_K):
        mx = jnp.max(work, axis=0, keepdims=True)
        hit = work == mx
        first = jnp.min(jnp.where(hit, iota, _M + 1), axis=0, keepdims=True)
        pick = iota == first
        sel = sel + pick.astype(jnp.float32)
        work = jnp.where(pick, -1.0, work)

    scale = 2.0 / _T
    a = re * sel * scale
    b = im * sel * scale
    dn_bwd = (((1,), (0,)), ((), ()))
    md = jax.lax.Precision.DEFAULT
    u = lax.dot_general(cb, a, dn_bwd, precision=md,
                        preferred_element_type=jnp.float32)
    v = lax.dot_general(sb, b, dn_bwd, precision=md,
                        preferred_element_type=jnp.float32)
    o_ref[0, :_TH, :] = (u - v)[:, :64]
    o_ref[0, _TH:, :] = (u + v)[:, :64]
    o_ref[1, :_TH, :] = (u - v)[:, 64:]
    o_ref[1, _TH:, :] = (u + v)[:, 64:]


@jax.jit
def kernel(x):
    b, t, d = x.shape
    xf = jnp.flip(x, axis=1)[:, : _T // 2, :]
    folded = pl.pallas_call(
        _body,
        grid=(b // 2,),
        in_specs=[
            pl.BlockSpec((2, _TH, 64), lambda i: (i, 0, 0)),
            pl.BlockSpec((2, _T // 2, 64), lambda i: (i, 0, 0)),
            pl.BlockSpec((_TH, _M), lambda i: (0, 0)),
            pl.BlockSpec((_TH, _M), lambda i: (0, 0)),
        ],
        out_specs=pl.BlockSpec((2, 2 * _TH, 64), lambda i: (i, 0, 0)),
        out_shape=jax.ShapeDtypeStruct((b, 2 * _TH, d), jnp.float32),
    )(x, xf, jnp.asarray(_C_TABLE), jnp.asarray(_S_TABLE))
    lo = folded[:, : _T // 2 + 1, :]
    hi = jnp.flip(folded[:, _TH + 1: _TH + _T // 2, :], axis=1)
    return jnp.concatenate([lo, hi, folded[:, : _PRED, :]], axis=1)


# speedup vs baseline: 1.5144x; 1.5144x over previous
import functools
import math

import jax
import jax.numpy as jnp
import numpy as np
from jax import lax
from jax.experimental import pallas as pl

_T = 2048
_PRED = 256
_K = 16
_M = 1024
_TH = 1032


def _basis():
    tau = np.arange(_T, dtype=np.int64)[:, None]
    m = np.arange(1, _M + 1, dtype=np.int64)[None, :]
    ang = 2.0 * np.pi * ((tau * m) % _T).astype(np.float64) / _T
    c = np.cos(ang)
    s = np.sin(ang)
    c[:, -1] = 0.0
    s[:, -1] = 0.0
    return c.astype(np.float32), s.astype(np.float32)


_C_TABLE, _S_TABLE = _basis()


def _body(x_ref, xf_ref, c_ref, s_ref, o_ref):
    xb = jnp.concatenate([x_ref[0], x_ref[1]], axis=1)
    xfb = jnp.concatenate([xf_ref[0], xf_ref[1]], axis=1)
    cb = c_ref[...]
    sb = s_ref[...]
    n = xb.shape[1]
    xr = xfb[0:1023, :]
    xmid = xb[1:1024, :]
    zpad = jnp.zeros((7, n), jnp.float32)
    zrow = jnp.zeros((1, n), jnp.float32)
    xe = jnp.concatenate([xb[0:1, :], xmid + xr, xb[1024:1025, :], zpad], 0)
    xo = jnp.concatenate([zrow, xmid - xr, zrow, zpad], 0)
    hi = jax.lax.Precision.HIGHEST
    dn_fwd = (((0,), (0,)), ((), ()))
    re = lax.dot_general(cb[:_TH, :], xe, dn_fwd, precision=hi,
                         preferred_element_type=jnp.float32)
    im = -lax.dot_general(sb[:_TH, :], xo, dn_fwd, precision=hi,
                          preferred_element_type=jnp.float32)
    mag2 = re * re + im * im

    iota = lax.broadcasted_iota(jnp.int32, (_M, n), 0)
    work = mag2
    sel = jnp.zeros(mag2.shape, jnp.float32)
    for _ in range(_K):
        mx = jnp.max(work, axis=0, keepdims=True)
        hit = work == mx
        first = jnp.min(jnp.where(hit, iota, _M + 1), axis=0, keepdims=True)
        pick = iota == first
        sel = sel + pick.astype(jnp.float32)
        work = jnp.where(pick, -1.0, work)

    scale = 2.0 / _T
    a = re * sel * scale
    b = im * sel * scale
    dn_bwd = (((1,), (0,)), ((), ()))
    md = jax.lax.Precision.DEFAULT
    rec = (lax.dot_general(cb, a, dn_bwd, precision=md,
                           preferred_element_type=jnp.float32)
           - lax.dot_general(sb, b, dn_bwd, precision=md,
                             preferred_element_type=jnp.float32))
    o_ref[0, : _T, :] = rec[:, :64]
    o_ref[0, _T:, :] = rec[: _PRED, :64]
    o_ref[1, : _T, :] = rec[:, 64:]
    o_ref[1, _T:, :] = rec[: _PRED, 64:]


@jax.jit
def kernel(x):
    b, t, d = x.shape
    xf = jnp.flip(x, axis=1)[:, : _T // 2, :]
    return pl.pallas_call(
        _body,
        grid=(b // 2,),
        in_specs=[
            pl.BlockSpec((2, _TH, 64), lambda i: (i, 0, 0)),
            pl.BlockSpec((2, _T // 2, 64), lambda i: (i, 0, 0)),
            pl.BlockSpec((_T, _M), lambda i: (0, 0)),
            pl.BlockSpec((_T, _M), lambda i: (0, 0)),
        ],
        out_specs=pl.BlockSpec((2, _T + _PRED, 64), lambda i: (i, 0, 0)),
        out_shape=jax.ShapeDtypeStruct((b, t + _PRED, d), jnp.float32),
    )(x, xf, jnp.asarray(_C_TABLE), jnp.asarray(_S_TABLE))
